# baseline (speedup 1.0000x reference)
import jax
import jax.numpy as jnp
from jax import lax
from jax.experimental import pallas as pl
from jax.experimental.pallas import tpu as pltpu
from jax.experimental.pallas import tpu_sc as plsc

L = 32768
K = 30
NC = 2
NS = 16
NW = NC * NS
ROWS_W = L // NW
CH = 128
CHUNKS = ROWS_W // CH
GRPS = CH // 16

_f32 = jnp.float32
_i32 = jnp.int32


def _sc_body(nc_hbm, dist_hbm, ang_hbm, seqp_hbm,
             idx_hbm, dis_hbm, ango_hbm,
             seqv,
             ncfA, distfA, angfA, ncfB, distfB, angfB,
             idxoA, disoA, angoA, idxoB, disoB, angoB,
             sinA, sinB, soutA, soutB):
    c = lax.axis_index("c")
    s = lax.axis_index("s")
    wid = s * NC + c
    row0 = wid * ROWS_W
    pltpu.sync_copy(seqp_hbm, seqv)
    lane = lax.iota(_i32, 16)
    zeros16 = jnp.zeros((16,), _f32)
    pad16 = jnp.full((16,), 22, _i32)
    ones16 = jnp.full((16,), 1, _i32)

    def in_copies(ci, ncf, distf, angf, sem):
        r0 = row0 + ci * CH
        return (
            pltpu.make_async_copy(nc_hbm.at[:, pl.ds(r0, CH)], ncf, sem),
            pltpu.make_async_copy(dist_hbm.at[:, pl.ds(r0, CH)], distf, sem),
            pltpu.make_async_copy(ang_hbm.at[:, :, pl.ds(r0, CH)], angf, sem),
        )

    def out_copies(ci, idxo, diso, ango, sem):
        r0 = row0 + ci * CH
        return (
            pltpu.make_async_copy(idxo, idx_hbm.at[:, pl.ds(r0, CH)], sem),
            pltpu.make_async_copy(diso, dis_hbm.at[pl.ds(r0 * 22, CH * 22)],
                                  sem),
            pltpu.make_async_copy(ango, ango_hbm.at[pl.ds(r0 * 132, CH * 132)],
                                  sem),
        )

    def msets(idxo, diso, ango):
        for si in range(22):
            for u in range(8):
                idxo[si, pl.ds(u * 16, 16)] = pad16
        for i in range(CH * 22 // 16):
            diso[pl.ds(i * 16, 16)] = zeros16
        for i in range(CH * 132 // 16):
            ango[pl.ds(i * 16, 16)] = zeros16

    def compute(ci, ncf, distf, angf, idxo, diso, ango):
        r0 = row0 + ci * CH

        def grp_body(g, cc):
            base = g * 16
            lrow = base + lane
            rowv = r0 + lrow
            base22 = lrow * 22
            base132 = lrow * 132

            def ldblk(k):
                v = ncf[k, pl.ds(base, 16)]
                dk = distf[k, pl.ds(base, 16)] * _f32(0.1)
                sw = plsc.load_gather(seqv, [jnp.right_shift(v, 2)])
                aks = tuple(angf[ch, k, pl.ds(base, 16)] * _f32(1.0 / 3.0)
                            for ch in range(6))
                d = v - rowv
                cond = jnp.abs(d) > 6
                inr = (d >= -6) & (d <= 6) & (d != 0)
                offi = jnp.where(inr, d + jnp.where(d < 0, 6, 5), 0)
                bit = jnp.left_shift(ones16, offi)
                sh = jnp.left_shift(v & 3, 3)
                sv = jnp.right_shift(sw, sh) & 0xFF
                return dk, aks, cond, inr, offi, bit, sv

            cnt = jnp.zeros((16,), _i32)
            fb = jnp.zeros((16,), _i32)
            pre = ldblk(0)
            for k in range(K):
                nxt = ldblk(k + 1) if k + 1 < K else None
                dk, aks, cond, inr, offi, bit, sv = pre
                w1 = cond & (cnt < 10)
                slot1 = jnp.minimum(cnt, 10)
                cnt = cnt + cond.astype(_i32)
                newf = inr & ((fb & bit) == 0)
                fb = fb | jnp.where(inr, bit, 0)
                wm = w1 | newf
                slot = jnp.where(w1, slot1, 10 + offi)
                plsc.store_scatter(idxo, [slot, lrow], sv, mask=wm)
                plsc.store_scatter(diso, [base22 + slot], dk, mask=wm)
                pa = base132 + slot * 6
                for ch in range(6):
                    plsc.store_scatter(ango, [pa + ch], aks[ch], mask=wm)
                pre = nxt
            return cc
        lax.fori_loop(0, GRPS, grp_body, 0)

    for cp in in_copies(0, ncfA, distfA, angfA, sinA):
        cp.start()

    def pair_body(j, carry):
        ci0 = 2 * j
        for cp in in_copies(ci0 + 1, ncfB, distfB, angfB, sinB):
            cp.start()
        for cp in in_copies(ci0, ncfA, distfA, angfA, sinA):
            cp.wait()

        @pl.when(j > 0)
        def _():
            for cp in out_copies(ci0 - 2, idxoA, disoA, angoA, soutA):
                cp.wait()
        msets(idxoA, disoA, angoA)
        compute(ci0, ncfA, distfA, angfA, idxoA, disoA, angoA)
        for cp in out_copies(ci0, idxoA, disoA, angoA, soutA):
            cp.start()

        @pl.when(j < CHUNKS // 2 - 1)
        def _():
            for cp in in_copies(ci0 + 2, ncfA, distfA, angfA, sinA):
                cp.start()
        for cp in in_copies(ci0 + 1, ncfB, distfB, angfB, sinB):
            cp.wait()

        @pl.when(j > 0)
        def _():
            for cp in out_copies(ci0 - 1, idxoB, disoB, angoB, soutB):
                cp.wait()
        msets(idxoB, disoB, angoB)
        compute(ci0 + 1, ncfB, distfB, angfB, idxoB, disoB, angoB)
        for cp in out_copies(ci0 + 1, idxoB, disoB, angoB, soutB):
            cp.start()
        return carry
    lax.fori_loop(0, CHUNKS // 2, pair_body, 0)

    for cp in out_copies(CHUNKS - 2, idxoA, disoA, angoA, soutA):
        cp.wait()
    for cp in out_copies(CHUNKS - 1, idxoB, disoB, angoB, soutB):
        cp.wait()


@jax.jit
def _sc_run(nc_t, dist_t, ang_t, seqp):
    mesh = plsc.VectorSubcoreMesh(core_axis_name="c", subcore_axis_name="s",
                                  num_cores=NC, num_subcores=NS)
    fn = pl.kernel(
        _sc_body,
        out_type=(
            jax.ShapeDtypeStruct((22, L), _i32),
            jax.ShapeDtypeStruct((L * 22,), _f32),
            jax.ShapeDtypeStruct((L * 132,), _f32),
        ),
        mesh=mesh,
        compiler_params=pltpu.CompilerParams(needs_layout_passes=False,
                                             use_tc_tiling_on_sc=False),
        scratch_types=[
            pltpu.VMEM((L // 4,), _i32),
            pltpu.VMEM((K, CH), _i32),
            pltpu.VMEM((K, CH), _f32),
            pltpu.VMEM((6, K, CH), _f32),
            pltpu.VMEM((K, CH), _i32),
            pltpu.VMEM((K, CH), _f32),
            pltpu.VMEM((6, K, CH), _f32),
            pltpu.VMEM((22, CH), _i32),
            pltpu.VMEM((CH * 22,), _f32),
            pltpu.VMEM((CH * 132,), _f32),
            pltpu.VMEM((22, CH), _i32),
            pltpu.VMEM((CH * 22,), _f32),
            pltpu.VMEM((CH * 132,), _f32),
            pltpu.SemaphoreType.DMA,
            pltpu.SemaphoreType.DMA,
            pltpu.SemaphoreType.DMA,
            pltpu.SemaphoreType.DMA,
        ],
    )
    return fn(nc_t, dist_t, ang_t, seqp)


def kernel(mask, num_cs, dist, angle, seqlist):
    Ln = mask.shape[0]
    nc_t = num_cs.astype(_i32).T
    dist_t = dist.T
    ang_t = angle.transpose(2, 1, 0)
    sq = seqlist.astype(_i32).reshape(-1, 4)
    seqp = (sq[:, 0] | jnp.left_shift(sq[:, 1], 8)
            | jnp.left_shift(sq[:, 2], 16) | jnp.left_shift(sq[:, 3], 24))
    idx_tr, dis_t, angle_t = _sc_run(nc_t, dist_t, ang_t, seqp)
    idx_t = idx_tr.T
    data_t = jnp.eye(23, dtype=_f32)
    label = seqlist.astype(_i32)
    return (data_t, idx_t, dis_t, angle_t, label, Ln)


# speedup vs baseline: 1.1692x; 1.1692x over previous
import jax
import jax.numpy as jnp
from jax import lax
from jax.experimental import pallas as pl
from jax.experimental.pallas import tpu as pltpu
from jax.experimental.pallas import tpu_sc as plsc

L = 32768
K = 30
NC = 2
NS = 16
NW = NC * NS
ROWS_W = L // NW
CH = 128
CHUNKS = ROWS_W // CH
GRPS = CH // 16

_f32 = jnp.float32
_i32 = jnp.int32


def _sc_body(nc_hbm, dist_hbm, ang_hbm, seqp_hbm,
             idx_hbm, dis_hbm, ango_hbm,
             seqv,
             ncfA, distfA, angfA, ncfB, distfB, angfB,
             idxoA, disoA, angoA, idxoB, disoB, angoB,
             sinA, sinB, soutA, soutB):
    c = lax.axis_index("c")
    s = lax.axis_index("s")
    wid = s * NC + c
    row0 = wid * ROWS_W
    pltpu.sync_copy(seqp_hbm, seqv)
    lane = lax.iota(_i32, 16)
    zeros16 = jnp.zeros((16,), _f32)
    pad16 = jnp.full((16,), 22, _i32)
    ones16 = jnp.full((16,), 1, _i32)

    def in_copies(ci, ncf, distf, angf, sem):
        r0 = row0 + ci * CH
        return (
            pltpu.make_async_copy(nc_hbm.at[:, pl.ds(r0, CH)], ncf, sem),
            pltpu.make_async_copy(dist_hbm.at[:, pl.ds(r0, CH)], distf, sem),
            pltpu.make_async_copy(ang_hbm.at[:, :, pl.ds(r0, CH)], angf, sem),
        )

    def out_copies(ci, idxo, diso, ango, sem):
        r0 = row0 + ci * CH
        return (
            pltpu.make_async_copy(idxo, idx_hbm.at[:, pl.ds(r0, CH)], sem),
            pltpu.make_async_copy(diso, dis_hbm.at[pl.ds(r0 * 22, CH * 22)],
                                  sem),
            pltpu.make_async_copy(ango, ango_hbm.at[pl.ds(r0 * 132, CH * 132)],
                                  sem),
        )

    def msets(idxo, diso, ango):
        def ms_idx(si, cc):
            for u in range(8):
                idxo[si, pl.ds(u * 16, 16)] = pad16
            return cc
        lax.fori_loop(0, 22, ms_idx, 0)

        def ms_dis(i, cc):
            b = i * 128
            for u in range(8):
                diso[pl.ds(b + u * 16, 16)] = zeros16
            return cc
        lax.fori_loop(0, CH * 22 // 128, ms_dis, 0)

        def ms_ang(i, cc):
            b = i * 128
            for u in range(8):
                ango[pl.ds(b + u * 16, 16)] = zeros16
            return cc
        lax.fori_loop(0, CH * 132 // 128, ms_ang, 0)

    def compute(ci, ncf, distf, angf, idxo, diso, ango):
        r0 = row0 + ci * CH

        def grp_body(g, cc):
            base = g * 16
            lrow = base + lane
            rowv = r0 + lrow
            base22 = lrow * 22
            base132 = lrow * 132

            def ldblk(k):
                v = ncf[k, pl.ds(base, 16)]
                dk = distf[k, pl.ds(base, 16)] * _f32(0.1)
                sw = plsc.load_gather(seqv, [jnp.right_shift(v, 2)])
                aks = tuple(angf[ch, k, pl.ds(base, 16)] * _f32(1.0 / 3.0)
                            for ch in range(6))
                d = v - rowv
                cond = jnp.abs(d) > 6
                inr = (d >= -6) & (d <= 6) & (d != 0)
                offi = jnp.where(inr, d + jnp.where(d < 0, 6, 5), 0)
                bit = jnp.left_shift(ones16, offi)
                sh = jnp.left_shift(v & 3, 3)
                sv = jnp.right_shift(sw, sh) & 0xFF
                return dk, aks, cond, inr, offi, bit, sv

            cnt = jnp.zeros((16,), _i32)
            fb = jnp.zeros((16,), _i32)
            pre = ldblk(0)
            for k in range(K):
                nxt = ldblk(k + 1) if k + 1 < K else None
                dk, aks, cond, inr, offi, bit, sv = pre
                w1 = cond & (cnt < 10)
                slot1 = jnp.minimum(cnt, 10)
                cnt = cnt + cond.astype(_i32)
                newf = inr & ((fb & bit) == 0)
                fb = fb | jnp.where(inr, bit, 0)
                wm = w1 | newf
                slot = jnp.where(w1, slot1, 10 + offi)
                plsc.store_scatter(idxo, [slot, lrow], sv, mask=wm)
                plsc.store_scatter(diso, [base22 + slot], dk, mask=wm)
                pa = base132 + slot * 6
                for ch in range(6):
                    plsc.store_scatter(ango, [pa + ch], aks[ch], mask=wm)
                pre = nxt
            return cc
        lax.fori_loop(0, GRPS, grp_body, 0)

    for cp in in_copies(0, ncfA, distfA, angfA, sinA):
        cp.start()

    def pair_body(j, carry):
        ci0 = 2 * j
        for cp in in_copies(ci0 + 1, ncfB, distfB, angfB, sinB):
            cp.start()
        for cp in in_copies(ci0, ncfA, distfA, angfA, sinA):
            cp.wait()

        @pl.when(j > 0)
        def _():
            for cp in out_copies(ci0 - 2, idxoA, disoA, angoA, soutA):
                cp.wait()
        msets(idxoA, disoA, angoA)
        compute(ci0, ncfA, distfA, angfA, idxoA, disoA, angoA)
        for cp in out_copies(ci0, idxoA, disoA, angoA, soutA):
            cp.start()

        @pl.when(j < CHUNKS // 2 - 1)
        def _():
            for cp in in_copies(ci0 + 2, ncfA, distfA, angfA, sinA):
                cp.start()
        for cp in in_copies(ci0 + 1, ncfB, distfB, angfB, sinB):
            cp.wait()

        @pl.when(j > 0)
        def _():
            for cp in out_copies(ci0 - 1, idxoB, disoB, angoB, soutB):
                cp.wait()
        msets(idxoB, disoB, angoB)
        compute(ci0 + 1, ncfB, distfB, angfB, idxoB, disoB, angoB)
        for cp in out_copies(ci0 + 1, idxoB, disoB, angoB, soutB):
            cp.start()
        return carry
    lax.fori_loop(0, CHUNKS // 2, pair_body, 0)

    for cp in out_copies(CHUNKS - 2, idxoA, disoA, angoA, soutA):
        cp.wait()
    for cp in out_copies(CHUNKS - 1, idxoB, disoB, angoB, soutB):
        cp.wait()


@jax.jit
def _sc_run(nc_t, dist_t, ang_t, seqp):
    mesh = plsc.VectorSubcoreMesh(core_axis_name="c", subcore_axis_name="s",
                                  num_cores=NC, num_subcores=NS)
    fn = pl.kernel(
        _sc_body,
        out_type=(
            jax.ShapeDtypeStruct((22, L), _i32),
            jax.ShapeDtypeStruct((L * 22,), _f32),
            jax.ShapeDtypeStruct((L * 132,), _f32),
        ),
        mesh=mesh,
        compiler_params=pltpu.CompilerParams(needs_layout_passes=False,
                                             use_tc_tiling_on_sc=False),
        scratch_types=[
            pltpu.VMEM((L // 4,), _i32),
            pltpu.VMEM((K, CH), _i32),
            pltpu.VMEM((K, CH), _f32),
            pltpu.VMEM((6, K, CH), _f32),
            pltpu.VMEM((K, CH), _i32),
            pltpu.VMEM((K, CH), _f32),
            pltpu.VMEM((6, K, CH), _f32),
            pltpu.VMEM((22, CH), _i32),
            pltpu.VMEM((CH * 22,), _f32),
            pltpu.VMEM((CH * 132,), _f32),
            pltpu.VMEM((22, CH), _i32),
            pltpu.VMEM((CH * 22,), _f32),
            pltpu.VMEM((CH * 132,), _f32),
            pltpu.SemaphoreType.DMA,
            pltpu.SemaphoreType.DMA,
            pltpu.SemaphoreType.DMA,
            pltpu.SemaphoreType.DMA,
        ],
    )
    return fn(nc_t, dist_t, ang_t, seqp)


def kernel(mask, num_cs, dist, angle, seqlist):
    Ln = mask.shape[0]
    nc_t = num_cs.astype(_i32).T
    dist_t = dist.T
    ang_t = angle.transpose(2, 1, 0)
    sq = seqlist.astype(_i32).reshape(-1, 4)
    seqp = (sq[:, 0] | jnp.left_shift(sq[:, 1], 8)
            | jnp.left_shift(sq[:, 2], 16) | jnp.left_shift(sq[:, 3], 24))
    idx_tr, dis_t, angle_t = _sc_run(nc_t, dist_t, ang_t, seqp)
    idx_t = idx_tr.T
    data_t = jnp.eye(23, dtype=_f32)
    label = seqlist.astype(_i32)
    return (data_t, idx_t, dis_t, angle_t, label, Ln)


# speedup vs baseline: 1.1739x; 1.0040x over previous
import jax
import jax.numpy as jnp
from jax import lax
from jax.experimental import pallas as pl
from jax.experimental.pallas import tpu as pltpu
from jax.experimental.pallas import tpu_sc as plsc

L = 32768
K = 30
NC = 2
NS = 16
NW = NC * NS
ROWS_W = L // NW
CH = 128
CHUNKS = ROWS_W // CH
GRPS = CH // 16

_f32 = jnp.float32
_i32 = jnp.int32


def _sc_body(nc_hbm, dist_hbm, ang_hbm, seqp_hbm,
             idx_hbm, dis_hbm, ango_hbm,
             seqv,
             ncfA, distfA, angfA, ncfB, distfB, angfB,
             idxoA, disoA, angoA, idxoB, disoB, angoB,
             sinA, sinB, soutA, soutB):
    c = lax.axis_index("c")
    s = lax.axis_index("s")
    wid = s * NC + c
    row0 = wid * ROWS_W
    pltpu.sync_copy(seqp_hbm, seqv)
    lane = lax.iota(_i32, 16)
    zeros16 = jnp.zeros((16,), _f32)
    pad16 = jnp.full((16,), 22, _i32)
    ones16 = jnp.full((16,), 1, _i32)

    def in_copies(ci, ncf, distf, angf, sem):
        r0 = row0 + ci * CH
        return (
            pltpu.make_async_copy(nc_hbm.at[:, pl.ds(r0, CH)], ncf, sem),
            pltpu.make_async_copy(dist_hbm.at[:, pl.ds(r0, CH)], distf, sem),
            pltpu.make_async_copy(ang_hbm.at[:, :, pl.ds(r0, CH)], angf, sem),
        )

    def out_copies(ci, idxo, diso, ango, sem):
        r0 = row0 + ci * CH
        return (
            pltpu.make_async_copy(idxo, idx_hbm.at[:, pl.ds(r0, CH)], sem),
            pltpu.make_async_copy(diso, dis_hbm.at[pl.ds(r0 * 22, CH * 22)],
                                  sem),
            pltpu.make_async_copy(ango, ango_hbm.at[pl.ds(r0 * 132, CH * 132)],
                                  sem),
        )

    def msets(idxo, diso, ango):
        def ms_idx(si, cc):
            for u in range(8):
                idxo[si, pl.ds(u * 16, 16)] = pad16
            return cc
        lax.fori_loop(0, 22, ms_idx, 0)

        def ms_dis(i, cc):
            b = i * 128
            for u in range(8):
                diso[pl.ds(b + u * 16, 16)] = zeros16
            return cc
        lax.fori_loop(0, CH * 22 // 128, ms_dis, 0)

        def ms_ang(i, cc):
            b = i * 128
            for u in range(8):
                ango[pl.ds(b + u * 16, 16)] = zeros16
            return cc
        lax.fori_loop(0, CH * 132 // 128, ms_ang, 0)

    def compute(ci, ncf, distf, angf, idxo, diso, ango):
        r0 = row0 + ci * CH

        def grp_body(g, cc):
            base = g * 16
            lrow = base + lane
            rowv = r0 + lrow
            base22 = lrow * 22
            base132 = lrow * 132

            def ldblk(k):
                v = ncf[k, pl.ds(base, 16)]
                dk = distf[k, pl.ds(base, 16)] * _f32(0.1)
                sw = plsc.load_gather(seqv, [jnp.right_shift(v, 2)])
                aks = tuple(angf[ch, k, pl.ds(base, 16)] * _f32(1.0 / 3.0)
                            for ch in range(6))
                d = v - rowv
                cond = jnp.abs(d) > 6
                inr = (d >= -6) & (d <= 6) & (d != 0)
                offi = jnp.where(inr, d + jnp.where(d < 0, 6, 5), 0)
                bit = jnp.left_shift(ones16, offi)
                sh = jnp.left_shift(v & 3, 3)
                sv = jnp.right_shift(sw, sh) & 0xFF
                return dk, aks, cond, inr, offi, bit, sv

            cnt = jnp.zeros((16,), _i32)
            fb = jnp.zeros((16,), _i32)
            pre = ldblk(0)
            for k in range(K):
                nxt = ldblk(k + 1) if k + 1 < K else None
                dk, aks, cond, inr, offi, bit, sv = pre
                w1 = cond & (cnt < 10)
                slot1 = jnp.minimum(cnt, 10)
                cnt = cnt + cond.astype(_i32)
                newf = inr & ((fb & bit) == 0)
                fb = fb | jnp.where(inr, bit, 0)
                wm = w1 | newf
                slot = jnp.where(w1, slot1, 10 + offi)
                plsc.store_scatter(idxo, [slot, lrow], sv, mask=wm)
                plsc.store_scatter(diso, [base22 + slot], dk, mask=wm)
                pa = base132 + slot * 6
                for ch in range(6):
                    plsc.store_scatter(ango, [pa + ch], aks[ch], mask=wm)
                pre = nxt
            return cc
        lax.fori_loop(0, GRPS, grp_body, 0)

    for cp in in_copies(0, ncfA, distfA, angfA, sinA):
        cp.start()

    def pair_body(j, carry):
        ci0 = 2 * j
        for cp in in_copies(ci0 + 1, ncfB, distfB, angfB, sinB):
            cp.start()
        for cp in in_copies(ci0, ncfA, distfA, angfA, sinA):
            cp.wait()

        @pl.when(j > 0)
        def _():
            for cp in out_copies(ci0 - 2, idxoA, disoA, angoA, soutA):
                cp.wait()
        msets(idxoA, disoA, angoA)
        compute(ci0, ncfA, distfA, angfA, idxoA, disoA, angoA)
        for cp in out_copies(ci0, idxoA, disoA, angoA, soutA):
            cp.start()

        @pl.when(j < CHUNKS // 2 - 1)
        def _():
            for cp in in_copies(ci0 + 2, ncfA, distfA, angfA, sinA):
                cp.start()
        for cp in in_copies(ci0 + 1, ncfB, distfB, angfB, sinB):
            cp.wait()

        @pl.when(j > 0)
        def _():
            for cp in out_copies(ci0 - 1, idxoB, disoB, angoB, soutB):
                cp.wait()
        msets(idxoB, disoB, angoB)
        compute(ci0 + 1, ncfB, distfB, angfB, idxoB, disoB, angoB)
        for cp in out_copies(ci0 + 1, idxoB, disoB, angoB, soutB):
            cp.start()
        return carry
    lax.fori_loop(0, CHUNKS // 2, pair_body, 0)

    for cp in out_copies(CHUNKS - 2, idxoA, disoA, angoA, soutA):
        cp.wait()
    for cp in out_copies(CHUNKS - 1, idxoB, disoB, angoB, soutB):
        cp.wait()


@jax.jit
def _sc_run(nc_t, dist_t, ang_t, seqp):
    mesh = plsc.VectorSubcoreMesh(core_axis_name="c", subcore_axis_name="s",
                                  num_cores=NC, num_subcores=NS)
    fn = pl.kernel(
        _sc_body,
        out_type=(
            jax.ShapeDtypeStruct((22, L), _i32),
            jax.ShapeDtypeStruct((L * 22,), _f32),
            jax.ShapeDtypeStruct((L * 132,), _f32),
        ),
        mesh=mesh,
        compiler_params=pltpu.CompilerParams(needs_layout_passes=False,
                                             use_tc_tiling_on_sc=False),
        scratch_types=[
            pltpu.VMEM((L // 4,), _i32),
            pltpu.VMEM((K, CH), _i32),
            pltpu.VMEM((K, CH), _f32),
            pltpu.VMEM((6, K, CH), _f32),
            pltpu.VMEM((K, CH), _i32),
            pltpu.VMEM((K, CH), _f32),
            pltpu.VMEM((6, K, CH), _f32),
            pltpu.VMEM((22, CH), _i32),
            pltpu.VMEM((CH * 22,), _f32),
            pltpu.VMEM((CH * 132,), _f32),
            pltpu.VMEM((22, CH), _i32),
            pltpu.VMEM((CH * 22,), _f32),
            pltpu.VMEM((CH * 132,), _f32),
            pltpu.SemaphoreType.DMA,
            pltpu.SemaphoreType.DMA,
            pltpu.SemaphoreType.DMA,
            pltpu.SemaphoreType.DMA,
        ],
    )
    return fn(nc_t, dist_t, ang_t, seqp)


def kernel(mask, num_cs, dist, angle, seqlist):
    Ln = mask.shape[0]
    nc_t = num_cs.astype(_i32).T
    dist_t = dist.T
    ang_t = angle.transpose(2, 1, 0)
    sq = seqlist.astype(_i32)
    seqp = (sq[0::4] | jnp.left_shift(sq[1::4], 8)
            | jnp.left_shift(sq[2::4], 16) | jnp.left_shift(sq[3::4], 24))
    idx_tr, dis_t, angle_t = _sc_run(nc_t, dist_t, ang_t, seqp)
    idx_t = idx_tr.T
    data_t = jnp.eye(23, dtype=_f32)
    label = seqlist.astype(_i32)
    return (data_t, idx_t, dis_t, angle_t, label, Ln)


# speedup vs baseline: 1.1750x; 1.0009x over previous
import jax
import jax.numpy as jnp
from jax import lax
from jax.experimental import pallas as pl
from jax.experimental.pallas import tpu as pltpu
from jax.experimental.pallas import tpu_sc as plsc

L = 32768
K = 30
NC = 2
NS = 16
NW = NC * NS
ROWS_W = L // NW
CH = 128
CHUNKS = ROWS_W // CH
GRPS = CH // 16

_f32 = jnp.float32
_i32 = jnp.int32


def _sc_body(nc_hbm, dist_hbm, ang_hbm, seqp_hbm,
             idx_hbm, dis_hbm, ango_hbm,
             seqv,
             ncfA, distfA, angfA, ncfB, distfB, angfB,
             idxoA, disoA, angoA, idxoB, disoB, angoB,
             sinA, sinB, soutA, soutB):
    c = lax.axis_index("c")
    s = lax.axis_index("s")
    wid = s * NC + c
    row0 = wid * ROWS_W
    pltpu.sync_copy(seqp_hbm, seqv)
    lane = lax.iota(_i32, 16)
    zeros16 = jnp.zeros((16,), _f32)
    pad16 = jnp.full((16,), 22, _i32)
    ones16 = jnp.full((16,), 1, _i32)

    def in_copies(ci, ncf, distf, angf, sem):
        r0 = row0 + ci * CH
        return (
            pltpu.make_async_copy(nc_hbm.at[:, pl.ds(r0, CH)], ncf, sem),
            pltpu.make_async_copy(dist_hbm.at[:, pl.ds(r0, CH)], distf, sem),
            pltpu.make_async_copy(ang_hbm.at[:, :, pl.ds(r0, CH)], angf, sem),
        )

    def out_copies(ci, idxo, diso, ango, sem):
        r0 = row0 + ci * CH
        return (
            pltpu.make_async_copy(idxo, idx_hbm.at[:, pl.ds(r0, CH)], sem),
            pltpu.make_async_copy(diso, dis_hbm.at[pl.ds(r0 * 22, CH * 22)],
                                  sem),
            pltpu.make_async_copy(ango, ango_hbm.at[pl.ds(r0 * 132, CH * 132)],
                                  sem),
        )

    def msets(idxo, diso, ango):
        def ms_idx(si, cc):
            for u in range(8):
                idxo[si, pl.ds(u * 16, 16)] = pad16
            return cc
        lax.fori_loop(0, 22, ms_idx, 0)

        def ms_dis(i, cc):
            b = i * 128
            for u in range(8):
                diso[pl.ds(b + u * 16, 16)] = zeros16
            return cc
        lax.fori_loop(0, CH * 22 // 128, ms_dis, 0)

        def ms_ang(i, cc):
            b = i * 128
            for u in range(8):
                ango[pl.ds(b + u * 16, 16)] = zeros16
            return cc
        lax.fori_loop(0, CH * 132 // 128, ms_ang, 0)

    def compute(ci, ncf, distf, angf, idxo, diso, ango):
        r0 = row0 + ci * CH

        def grp_body(g, cc):
            base = g * 16
            lrow = base + lane
            rowv = r0 + lrow
            base22 = lrow * 22
            base132 = lrow * 132

            def ldblk(k):
                v = ncf[k, pl.ds(base, 16)]
                dk = distf[k, pl.ds(base, 16)] * _f32(0.1)
                sw = plsc.load_gather(seqv, [jnp.right_shift(v, 1)])
                aks = tuple(angf[ch, k, pl.ds(base, 16)] * _f32(1.0 / 3.0)
                            for ch in range(6))
                d = v - rowv
                cond = jnp.abs(d) > 6
                inr = (d >= -6) & (d <= 6) & (d != 0)
                offi = jnp.where(inr, d + jnp.where(d < 0, 6, 5), 0)
                bit = jnp.left_shift(ones16, offi)
                sh = jnp.left_shift(v & 1, 4)
                sv = jnp.right_shift(sw, sh) & 0xFFFF
                return dk, aks, cond, inr, offi, bit, sv

            cnt = jnp.zeros((16,), _i32)
            fb = jnp.zeros((16,), _i32)
            pre = ldblk(0)
            for k in range(K):
                nxt = ldblk(k + 1) if k + 1 < K else None
                dk, aks, cond, inr, offi, bit, sv = pre
                w1 = cond & (cnt < 10)
                slot1 = jnp.minimum(cnt, 10)
                cnt = cnt + cond.astype(_i32)
                newf = inr & ((fb & bit) == 0)
                fb = fb | jnp.where(inr, bit, 0)
                wm = w1 | newf
                slot = jnp.where(w1, slot1, 10 + offi)
                plsc.store_scatter(idxo, [slot, lrow], sv, mask=wm)
                plsc.store_scatter(diso, [base22 + slot], dk, mask=wm)
                pa = base132 + slot * 6
                for ch in range(6):
                    plsc.store_scatter(ango, [pa + ch], aks[ch], mask=wm)
                pre = nxt
            return cc
        lax.fori_loop(0, GRPS, grp_body, 0)

    for cp in in_copies(0, ncfA, distfA, angfA, sinA):
        cp.start()

    def pair_body(j, carry):
        ci0 = 2 * j
        for cp in in_copies(ci0 + 1, ncfB, distfB, angfB, sinB):
            cp.start()
        for cp in in_copies(ci0, ncfA, distfA, angfA, sinA):
            cp.wait()

        @pl.when(j > 0)
        def _():
            for cp in out_copies(ci0 - 2, idxoA, disoA, angoA, soutA):
                cp.wait()
        msets(idxoA, disoA, angoA)
        compute(ci0, ncfA, distfA, angfA, idxoA, disoA, angoA)
        for cp in out_copies(ci0, idxoA, disoA, angoA, soutA):
            cp.start()

        @pl.when(j < CHUNKS // 2 - 1)
        def _():
            for cp in in_copies(ci0 + 2, ncfA, distfA, angfA, sinA):
                cp.start()
        for cp in in_copies(ci0 + 1, ncfB, distfB, angfB, sinB):
            cp.wait()

        @pl.when(j > 0)
        def _():
            for cp in out_copies(ci0 - 1, idxoB, disoB, angoB, soutB):
                cp.wait()
        msets(idxoB, disoB, angoB)
        compute(ci0 + 1, ncfB, distfB, angfB, idxoB, disoB, angoB)
        for cp in out_copies(ci0 + 1, idxoB, disoB, angoB, soutB):
            cp.start()
        return carry
    lax.fori_loop(0, CHUNKS // 2, pair_body, 0)

    for cp in out_copies(CHUNKS - 2, idxoA, disoA, angoA, soutA):
        cp.wait()
    for cp in out_copies(CHUNKS - 1, idxoB, disoB, angoB, soutB):
        cp.wait()


@jax.jit
def _sc_run(nc_t, dist_t, ang_t, seqp):
    mesh = plsc.VectorSubcoreMesh(core_axis_name="c", subcore_axis_name="s",
                                  num_cores=NC, num_subcores=NS)
    fn = pl.kernel(
        _sc_body,
        out_type=(
            jax.ShapeDtypeStruct((22, L), _i32),
            jax.ShapeDtypeStruct((L * 22,), _f32),
            jax.ShapeDtypeStruct((L * 132,), _f32),
        ),
        mesh=mesh,
        compiler_params=pltpu.CompilerParams(needs_layout_passes=False,
                                             use_tc_tiling_on_sc=False),
        scratch_types=[
            pltpu.VMEM((L // 2,), _i32),
            pltpu.VMEM((K, CH), _i32),
            pltpu.VMEM((K, CH), _f32),
            pltpu.VMEM((6, K, CH), _f32),
            pltpu.VMEM((K, CH), _i32),
            pltpu.VMEM((K, CH), _f32),
            pltpu.VMEM((6, K, CH), _f32),
            pltpu.VMEM((22, CH), _i32),
            pltpu.VMEM((CH * 22,), _f32),
            pltpu.VMEM((CH * 132,), _f32),
            pltpu.VMEM((22, CH), _i32),
            pltpu.VMEM((CH * 22,), _f32),
            pltpu.VMEM((CH * 132,), _f32),
            pltpu.SemaphoreType.DMA,
            pltpu.SemaphoreType.DMA,
            pltpu.SemaphoreType.DMA,
            pltpu.SemaphoreType.DMA,
        ],
    )
    return fn(nc_t, dist_t, ang_t, seqp)


def kernel(mask, num_cs, dist, angle, seqlist):
    Ln = mask.shape[0]
    nc_t = num_cs.astype(_i32).T
    dist_t = dist.T
    ang_t = angle.transpose(2, 1, 0)
    sq = seqlist.astype(_i32)
    seqp = sq[0::2] | jnp.left_shift(sq[1::2], 16)
    idx_tr, dis_t, angle_t = _sc_run(nc_t, dist_t, ang_t, seqp)
    idx_t = idx_tr.T
    data_t = jnp.eye(23, dtype=_f32)
    label = seqlist.astype(_i32)
    return (data_t, idx_t, dis_t, angle_t, label, Ln)
